# baseline (speedup 1.0000x reference)
import functools

import jax
import jax.numpy as jnp
from jax import lax
from jax.experimental import pallas as pl
from jax.experimental.pallas import tpu as pltpu
from jax.experimental.pallas import tpu_sc as plsc

_NROWS = 1_000_000
_NZ = 100
_R = 320
_CW = _R * _NZ
_NCH = _NROWS // _R
_NSEM = 8

_info = plsc.get_sparse_core_info()
_NC = _info.num_cores
_NW = _NC * _info.num_subcores
_NT_HI = -(-_NCH // _NW)
_NT_LO = _NCH // _NW
_N_HI = _NCH - _NT_LO * _NW
_ZZW = _NT_HI * _R
_NGRP = -(-_ZZW // 128)


@functools.partial(
    pl.kernel,
    out_type=jax.ShapeDtypeStruct((_NROWS * _NZ,), jnp.float32),
    mesh=plsc.VectorSubcoreMesh(core_axis_name="c", subcore_axis_name="s"),
    compiler_params=pltpu.CompilerParams(needs_layout_passes=False),
    scratch_types=[
        pltpu.VMEM((128,), jnp.int32),
        pltpu.VMEM((_ZZW,), jnp.int32),
        pltpu.VMEM((_CW,), jnp.float32),
        pltpu.VMEM_SHARED((_CW,), jnp.float32),
        pltpu.VMEM((_NGRP, 128), jnp.int32),
        pltpu.VMEM((128,), jnp.float32),
        pltpu.SemaphoreType.DMA,
        pltpu.SemaphoreType.DMA,
        pltpu.SemaphoreType.DMA,
        pltpu.SemaphoreType.DMA,
        pltpu.SemaphoreType.DMA,
        pltpu.SemaphoreType.DMA,
        pltpu.SemaphoreType.DMA,
        pltpu.SemaphoreType.DMA,
    ],
)
def _sc_onehot(zz_hbm, zmap_hbm, out_hbm,
               zmap_v, zz_v, zv, spz, opos, ones_v,
               s0, s1, s2, s3, s4, s5, s6, s7):
    sems = (s0, s1, s2, s3, s4, s5, s6, s7)
    cid = lax.axis_index("c")
    sid = lax.axis_index("s")
    w = sid * _NC + cid

    zeros16 = jnp.zeros((16,), jnp.float32)
    ones16 = jnp.ones((16,), jnp.float32)
    iota16 = lax.iota(jnp.int32, 16)

    @pl.when(sid == 0)
    def _():
        def _z(i, _):
            zv[pl.ds(i * 16, 16)] = zeros16
            return 0
        lax.fori_loop(0, _CW // 16, _z, 0)
        pltpu.sync_copy(zv, spz)
    plsc.subcore_barrier()

    pltpu.sync_copy(zmap_hbm, zmap_v)
    for k in range(8):
        ones_v[pl.ds(k * 16, 16)] = ones16

    is_hi = w < _N_HI
    start = jnp.where(is_hi, _NT_HI * w, _NT_LO * w + _N_HI)
    nt = jnp.where(is_hi, _NT_HI, _NT_LO)
    lo_words = _NT_LO * _R

    def _zq(q, _):
        for b in range(_NSEM):
            t = q * _NSEM + b

            @pl.when(t < nt)
            def _():
                dst = out_hbm.at[pl.ds((start + t) * _CW, _CW)]

                @pl.when(t >= _NSEM)
                def _():
                    pltpu.make_async_copy(spz, dst, sems[b]).wait()
                pltpu.make_async_copy(spz, dst, sems[b]).start()
        return 0
    lax.fori_loop(0, -(-(_NT_HI) // _NSEM), _zq, 0)

    pltpu.sync_copy(zz_hbm.at[pl.ds(start * _R, lo_words)],
                    zz_v.at[pl.ds(0, lo_words)])

    @pl.when(is_hi)
    def _():
        pltpu.sync_copy(zz_hbm.at[pl.ds(start * _R + lo_words, _R)],
                        zz_v.at[pl.ds(lo_words, _R)])

    idx0 = plsc.load_gather(zmap_v, [zz_v[pl.ds(0, 16)]])
    opos0 = (start * _R + iota16) * _NZ + idx0
    for g in range(_ZZW // (128 * 16) * 16, _NGRP):
        for k in range(8):
            opos[g, pl.ds(k * 16, 16)] = opos0

    def _grp(j, _):
        z = zz_v[pl.ds(j * 16, 16)]
        idx = plsc.load_gather(zmap_v, [z])
        op = (start * _R + j * 16 + iota16) * _NZ + idx
        opos[j // 8, pl.ds((j % 8) * 16, 16)] = op
        return 0
    lax.fori_loop(0, nt * (_R // 16), _grp, 0)

    for b in range(_NSEM):
        pltpu.make_async_copy(spz, out_hbm.at[pl.ds(0, _CW)], sems[b]).wait()

    ngrp = (nt * _R + 127) // 128

    def _oq(q, _):
        for b in range(_NSEM):
            g = q * _NSEM + b

            @pl.when(g < ngrp)
            def _():
                @pl.when(g >= _NSEM)
                def _():
                    pltpu.make_async_copy(
                        ones_v, out_hbm.at[opos.at[0]], sems[b]).wait()
                pltpu.make_async_copy(
                    ones_v, out_hbm.at[opos.at[g]], sems[b]).start()
        return 0
    lax.fori_loop(0, -(-_NGRP // _NSEM), _oq, 0)

    for b in range(_NSEM):
        pltpu.make_async_copy(ones_v, out_hbm.at[opos.at[0]], sems[b]).wait()


def kernel(zz, zmap):
    zz = zz.reshape(-1).astype(jnp.int32)
    zmap_p = jnp.zeros((128,), jnp.int32).at[:_NZ].set(zmap.astype(jnp.int32))
    out = _sc_onehot(zz, zmap_p)
    return out.reshape(_NROWS, _NZ)


# speedup vs baseline: 2.4540x; 2.4540x over previous
import functools

import jax
import jax.numpy as jnp
from jax import lax
from jax.experimental import pallas as pl
from jax.experimental.pallas import tpu as pltpu
from jax.experimental.pallas import tpu_sc as plsc

_NROWS = 1_000_000
_NZ = 100
_R = 400
_NCH = _NROWS // _R
_G = _R // 16
_NPAD = 1 << 20

_info = plsc.get_sparse_core_info()
_NC = _info.num_cores
_NW = _NC * _info.num_subcores
_NT_HI = -(-_NCH // _NW)
_NT_LO = _NCH // _NW
_N_HI = _NCH - _NT_LO * _NW
_ZZW = _NT_HI * _R


@functools.partial(
    pl.kernel,
    out_type=jax.ShapeDtypeStruct((_NPAD,), jnp.int32),
    mesh=plsc.VectorSubcoreMesh(core_axis_name="c", subcore_axis_name="s"),
    compiler_params=pltpu.CompilerParams(needs_layout_passes=False),
    scratch_types=[
        pltpu.VMEM((128,), jnp.int32),
        pltpu.VMEM((_ZZW,), jnp.int32),
        pltpu.VMEM((_ZZW,), jnp.int32),
    ],
)
def _sc_gather(zz_hbm, zmap_hbm, idx_hbm, zmap_v, zz_v, ibuf):
    w = lax.axis_index("s") * _NC + lax.axis_index("c")

    pltpu.sync_copy(zmap_hbm, zmap_v)

    is_hi = w < _N_HI
    start = jnp.where(is_hi, _NT_HI * w, _NT_LO * w + _N_HI)
    lo_words = _NT_LO * _R

    pltpu.sync_copy(zz_hbm.at[pl.ds(start * _R, lo_words)],
                    zz_v.at[pl.ds(0, lo_words)])

    @pl.when(is_hi)
    def _():
        pltpu.sync_copy(zz_hbm.at[pl.ds(start * _R + lo_words, _R)],
                        zz_v.at[pl.ds(lo_words, _R)])

    def _group(j, _):
        z = zz_v[pl.ds(j * 16, 16)]
        ibuf[pl.ds(j * 16, 16)] = plsc.load_gather(zmap_v, [z])
        return 0
    lax.fori_loop(0, lo_words // 16, _group, 0)

    @pl.when(is_hi)
    def _():
        for g in range(_G):
            z = zz_v[pl.ds(lo_words + g * 16, 16)]
            ibuf[pl.ds(lo_words + g * 16, 16)] = plsc.load_gather(zmap_v, [z])

    pltpu.sync_copy(ibuf.at[pl.ds(0, lo_words)],
                    idx_hbm.at[pl.ds(start * _R, lo_words)])

    @pl.when(is_hi)
    def _():
        pltpu.sync_copy(ibuf.at[pl.ds(lo_words, _R)],
                        idx_hbm.at[pl.ds(start * _R + lo_words, _R)])


_BLKR = 1024
_TGRID = -(-_NROWS // _BLKR)
_TAIL = _NROWS - (_TGRID - 1) * _BLKR
_NBUF = 8


def _tc_body(idx_ref, out_ref):
    nsub = _BLKR // 128
    uf = idx_ref[...].astype(jnp.float32)
    lane_iota = lax.broadcasted_iota(jnp.int32, (128, 128), 1).astype(jnp.float32)
    sub_iota = lax.broadcasted_iota(jnp.int32, (nsub, 128), 0)
    for s in range(nsub):
        sel = (sub_iota == s).astype(jnp.float32)
        bc = lax.dot_general(uf, sel, (((0,), (0,)), ((), ())),
                             preferred_element_type=jnp.float32)
        oh = (bc == lane_iota).astype(jnp.float32)
        out_ref[pl.ds(s * 128, 128), :] = oh


def kernel(zz, zmap):
    zz = zz.reshape(-1).astype(jnp.int32)
    zmap_p = jnp.zeros((128,), jnp.int32).at[:_NZ].set(zmap.astype(jnp.int32))
    idx = _sc_gather(zz, zmap_p)
    out = pl.pallas_call(
        _tc_body,
        grid=(_TGRID,),
        in_specs=[pl.BlockSpec((_BLKR // 128, 128), lambda i: (i, 0))],
        out_specs=pl.BlockSpec((_BLKR, 128), lambda i: (i, 0)),
        out_shape=jax.ShapeDtypeStruct((_NROWS, 128), jnp.float32),
    )(idx.reshape(_NPAD // 128, 128))
    return out[:, :_NZ]


# speedup vs baseline: 3.4804x; 1.4183x over previous
import functools

import jax
import jax.numpy as jnp
from jax import lax
from jax.experimental import pallas as pl
from jax.experimental.pallas import tpu as pltpu
from jax.experimental.pallas import tpu_sc as plsc

_NROWS = 1_000_000
_NZ = 100
_R = 400
_NCH = _NROWS // _R
_G = _R // 16
_NPAD = 1 << 20

_info = plsc.get_sparse_core_info()
_NC = _info.num_cores
_NW = _NC * _info.num_subcores
_NT_HI = -(-_NCH // _NW)
_NT_LO = _NCH // _NW
_N_HI = _NCH - _NT_LO * _NW
_ZZW = _NT_HI * _R


@functools.partial(
    pl.kernel,
    out_type=jax.ShapeDtypeStruct((_NPAD,), jnp.int32),
    mesh=plsc.VectorSubcoreMesh(core_axis_name="c", subcore_axis_name="s"),
    compiler_params=pltpu.CompilerParams(needs_layout_passes=False),
    scratch_types=[
        pltpu.VMEM((128,), jnp.int32),
        pltpu.VMEM((_ZZW,), jnp.int32),
        pltpu.VMEM((_ZZW,), jnp.int32),
    ],
)
def _sc_gather(zz_hbm, zmap_hbm, idx_hbm, zmap_v, zz_v, ibuf):
    w = lax.axis_index("s") * _NC + lax.axis_index("c")

    pltpu.sync_copy(zmap_hbm, zmap_v)

    is_hi = w < _N_HI
    start = jnp.where(is_hi, _NT_HI * w, _NT_LO * w + _N_HI)
    lo_words = _NT_LO * _R

    pltpu.sync_copy(zz_hbm.at[pl.ds(start * _R, lo_words)],
                    zz_v.at[pl.ds(0, lo_words)])

    @pl.when(is_hi)
    def _():
        pltpu.sync_copy(zz_hbm.at[pl.ds(start * _R + lo_words, _R)],
                        zz_v.at[pl.ds(lo_words, _R)])

    def _group(j, _):
        z = zz_v[pl.ds(j * 16, 16)]
        ibuf[pl.ds(j * 16, 16)] = plsc.load_gather(zmap_v, [z])
        return 0
    lax.fori_loop(0, lo_words // 16, _group, 0)

    @pl.when(is_hi)
    def _():
        for g in range(_G):
            z = zz_v[pl.ds(lo_words + g * 16, 16)]
            ibuf[pl.ds(lo_words + g * 16, 16)] = plsc.load_gather(zmap_v, [z])

    pltpu.sync_copy(ibuf.at[pl.ds(0, lo_words)],
                    idx_hbm.at[pl.ds(start * _R, lo_words)])

    @pl.when(is_hi)
    def _():
        pltpu.sync_copy(ibuf.at[pl.ds(lo_words, _R)],
                        idx_hbm.at[pl.ds(start * _R + lo_words, _R)])


_BLKR = 2048
_TGRID = -(-_NROWS // _BLKR)
_TAIL = _NROWS - (_TGRID - 1) * _BLKR
_NBUF = 8


def _tc_body(idx_ref, out_hbm, buf, sems):
    nsub = _BLKR // 128
    i = pl.program_id(0)
    j = lax.rem(i, _NBUF)
    uf = idx_ref[...].astype(jnp.float32)
    lane_iota = lax.broadcasted_iota(jnp.int32, (128, _NZ), 1).astype(jnp.float32)
    sub_iota = lax.broadcasted_iota(jnp.int32, (nsub, _NZ), 0)

    @pl.when(i >= _NBUF)
    def _():
        pltpu.make_async_copy(buf.at[j], out_hbm.at[pl.ds(0, _BLKR), :],
                              sems.at[j]).wait()

    for s in range(nsub):
        sel = (sub_iota == s).astype(jnp.float32)
        bc = lax.dot_general(uf, sel, (((0,), (0,)), ((), ())),
                             preferred_element_type=jnp.float32)
        oh = (bc == lane_iota).astype(jnp.float32)
        buf[j, pl.ds(s * 128, 128), :] = oh

    @pl.when(i < _TGRID - 1)
    def _():
        pltpu.make_async_copy(buf.at[j],
                              out_hbm.at[pl.ds(i * _BLKR, _BLKR), :],
                              sems.at[j]).start()

    @pl.when(i == _TGRID - 1)
    def _():
        jl = (_TGRID - 1) % _NBUF
        tail = pltpu.make_async_copy(
            buf.at[jl, pl.ds(0, _TAIL), :],
            out_hbm.at[pl.ds((_TGRID - 1) * _BLKR, _TAIL), :],
            sems.at[jl])
        tail.start()
        tail.wait()
        for k in range(_NBUF):
            if k != jl:
                pltpu.make_async_copy(buf.at[k],
                                      out_hbm.at[pl.ds(0, _BLKR), :],
                                      sems.at[k]).wait()


def kernel(zz, zmap):
    zz = zz.reshape(-1).astype(jnp.int32)
    zmap_p = jnp.zeros((128,), jnp.int32).at[:_NZ].set(zmap.astype(jnp.int32))
    idx = _sc_gather(zz, zmap_p)
    out = pl.pallas_call(
        _tc_body,
        grid=(_TGRID,),
        in_specs=[pl.BlockSpec((_BLKR // 128, 128), lambda i: (i, 0))],
        out_specs=pl.BlockSpec(memory_space=pltpu.HBM),
        out_shape=jax.ShapeDtypeStruct((_NROWS, _NZ), jnp.float32),
        scratch_shapes=[
            pltpu.VMEM((_NBUF, _BLKR, _NZ), jnp.float32),
            pltpu.SemaphoreType.DMA((_NBUF,)),
        ],
    )(idx.reshape(_NPAD // 128, 128))
    return out


# speedup vs baseline: 4.0406x; 1.1610x over previous
import functools

import jax
import jax.numpy as jnp
from jax import lax
from jax.experimental import pallas as pl
from jax.experimental.pallas import tpu as pltpu
from jax.experimental.pallas import tpu_sc as plsc

_NROWS = 1_000_000
_NZ = 100
_R = 400
_NCH = _NROWS // _R
_G = _R // 16
_NPAD = 1 << 20

_info = plsc.get_sparse_core_info()
_NC = _info.num_cores
_NW = _NC * _info.num_subcores
_NT_HI = -(-_NCH // _NW)
_NT_LO = _NCH // _NW
_N_HI = _NCH - _NT_LO * _NW
_ZZW = _NT_HI * _R


@functools.partial(
    pl.kernel,
    out_type=jax.ShapeDtypeStruct((_NPAD,), jnp.int32),
    mesh=plsc.VectorSubcoreMesh(core_axis_name="c", subcore_axis_name="s"),
    compiler_params=pltpu.CompilerParams(needs_layout_passes=False),
    scratch_types=[
        pltpu.VMEM((128,), jnp.int32),
        pltpu.VMEM((_ZZW,), jnp.int32),
        pltpu.VMEM((_ZZW,), jnp.int32),
    ],
)
def _sc_gather(zz_hbm, zmap_hbm, idx_hbm, zmap_v, zz_v, ibuf):
    w = lax.axis_index("s") * _NC + lax.axis_index("c")

    pltpu.sync_copy(zmap_hbm, zmap_v)

    is_hi = w < _N_HI
    start = jnp.where(is_hi, _NT_HI * w, _NT_LO * w + _N_HI)
    lo_words = _NT_LO * _R

    pltpu.sync_copy(zz_hbm.at[pl.ds(start * _R, lo_words)],
                    zz_v.at[pl.ds(0, lo_words)])

    @pl.when(is_hi)
    def _():
        pltpu.sync_copy(zz_hbm.at[pl.ds(start * _R + lo_words, _R)],
                        zz_v.at[pl.ds(lo_words, _R)])

    def _group(j, _):
        z = zz_v[pl.ds(j * 16, 16)]
        ibuf[pl.ds(j * 16, 16)] = plsc.load_gather(zmap_v, [z])
        return 0
    lax.fori_loop(0, lo_words // 16, _group, 0)

    @pl.when(is_hi)
    def _():
        for g in range(_G):
            z = zz_v[pl.ds(lo_words + g * 16, 16)]
            ibuf[pl.ds(lo_words + g * 16, 16)] = plsc.load_gather(zmap_v, [z])

    pltpu.sync_copy(ibuf.at[pl.ds(0, lo_words)],
                    idx_hbm.at[pl.ds(start * _R, lo_words)])

    @pl.when(is_hi)
    def _():
        pltpu.sync_copy(ibuf.at[pl.ds(lo_words, _R)],
                        idx_hbm.at[pl.ds(start * _R + lo_words, _R)])


_BLKR = 4096
_TGRID = -(-_NROWS // _BLKR)
_TAIL = _NROWS - (_TGRID - 1) * _BLKR
_NBUF = 8


def _tc_body(idx_ref, out_hbm, buf, sems):
    nsub = _BLKR // 128
    i = pl.program_id(0)
    j = lax.rem(i, _NBUF)
    uf = idx_ref[...].astype(jnp.float32)
    lane_iota = lax.broadcasted_iota(jnp.int32, (128, _NZ), 1).astype(jnp.float32)
    sub_iota = lax.broadcasted_iota(jnp.int32, (nsub, _NZ), 0)

    @pl.when(i >= _NBUF)
    def _():
        pltpu.make_async_copy(buf.at[j], out_hbm.at[pl.ds(0, _BLKR), :],
                              sems.at[j]).wait()

    for s in range(nsub):
        sel = (sub_iota == s).astype(jnp.float32)
        bc = lax.dot_general(uf, sel, (((0,), (0,)), ((), ())),
                             preferred_element_type=jnp.float32)
        oh = (bc == lane_iota).astype(jnp.float32)
        buf[j, pl.ds(s * 128, 128), :] = oh

    @pl.when(i < _TGRID - 1)
    def _():
        pltpu.make_async_copy(buf.at[j],
                              out_hbm.at[pl.ds(i * _BLKR, _BLKR), :],
                              sems.at[j]).start()

    @pl.when(i == _TGRID - 1)
    def _():
        jl = (_TGRID - 1) % _NBUF
        tail = pltpu.make_async_copy(
            buf.at[jl, pl.ds(0, _TAIL), :],
            out_hbm.at[pl.ds((_TGRID - 1) * _BLKR, _TAIL), :],
            sems.at[jl])
        tail.start()
        tail.wait()
        for k in range(_NBUF):
            if k != jl:
                pltpu.make_async_copy(buf.at[k],
                                      out_hbm.at[pl.ds(0, _BLKR), :],
                                      sems.at[k]).wait()


def kernel(zz, zmap):
    zz = zz.reshape(-1).astype(jnp.int32)
    zmap_p = jnp.zeros((128,), jnp.int32).at[:_NZ].set(zmap.astype(jnp.int32))
    idx = _sc_gather(zz, zmap_p)
    out = pl.pallas_call(
        _tc_body,
        grid=(_TGRID,),
        in_specs=[pl.BlockSpec((_BLKR // 128, 128), lambda i: (i, 0))],
        out_specs=pl.BlockSpec(memory_space=pltpu.HBM),
        out_shape=jax.ShapeDtypeStruct((_NROWS, _NZ), jnp.float32),
        scratch_shapes=[
            pltpu.VMEM((_NBUF, _BLKR, _NZ), jnp.float32),
            pltpu.SemaphoreType.DMA((_NBUF,)),
        ],
    )(idx.reshape(_NPAD // 128, 128))
    return out


# speedup vs baseline: 4.2709x; 1.0570x over previous
import functools

import jax
import jax.numpy as jnp
from jax import lax
from jax.experimental import pallas as pl
from jax.experimental.pallas import tpu as pltpu
from jax.experimental.pallas import tpu_sc as plsc

_NROWS = 1_000_000
_NZ = 100
_R = 400
_NCH = _NROWS // _R
_G = _R // 16
_NPAD = 1 << 20

_info = plsc.get_sparse_core_info()
_NC = _info.num_cores
_NW = _NC * _info.num_subcores
_NT_HI = -(-_NCH // _NW)
_NT_LO = _NCH // _NW
_N_HI = _NCH - _NT_LO * _NW
_ZZW = _NT_HI * _R


@functools.partial(
    pl.kernel,
    out_type=jax.ShapeDtypeStruct((_NPAD,), jnp.int32),
    mesh=plsc.VectorSubcoreMesh(core_axis_name="c", subcore_axis_name="s"),
    compiler_params=pltpu.CompilerParams(needs_layout_passes=False),
    scratch_types=[
        pltpu.VMEM((128,), jnp.int32),
        pltpu.VMEM((_ZZW,), jnp.int32),
        pltpu.VMEM((_ZZW,), jnp.int32),
    ],
)
def _sc_gather(zz_hbm, zmap_hbm, idx_hbm, zmap_v, zz_v, ibuf):
    w = lax.axis_index("s") * _NC + lax.axis_index("c")

    pltpu.sync_copy(zmap_hbm, zmap_v)

    is_hi = w < _N_HI
    start = jnp.where(is_hi, _NT_HI * w, _NT_LO * w + _N_HI)
    lo_words = _NT_LO * _R

    pltpu.sync_copy(zz_hbm.at[pl.ds(start * _R, lo_words)],
                    zz_v.at[pl.ds(0, lo_words)])

    @pl.when(is_hi)
    def _():
        pltpu.sync_copy(zz_hbm.at[pl.ds(start * _R + lo_words, _R)],
                        zz_v.at[pl.ds(lo_words, _R)])

    def _group(j, _):
        z = zz_v[pl.ds(j * 16, 16)]
        ibuf[pl.ds(j * 16, 16)] = plsc.load_gather(zmap_v, [z])
        return 0
    lax.fori_loop(0, lo_words // 16, _group, 0)

    @pl.when(is_hi)
    def _():
        for g in range(_G):
            z = zz_v[pl.ds(lo_words + g * 16, 16)]
            ibuf[pl.ds(lo_words + g * 16, 16)] = plsc.load_gather(zmap_v, [z])

    pltpu.sync_copy(ibuf.at[pl.ds(0, lo_words)],
                    idx_hbm.at[pl.ds(start * _R, lo_words)])

    @pl.when(is_hi)
    def _():
        pltpu.sync_copy(ibuf.at[pl.ds(lo_words, _R)],
                        idx_hbm.at[pl.ds(start * _R + lo_words, _R)])


_BLKR = 8192
_TGRID = -(-_NROWS // _BLKR)
_TAIL = _NROWS - (_TGRID - 1) * _BLKR
_NBUF = 8


def _tc_body(idx_ref, out_hbm, buf, sems):
    nsub = _BLKR // 128
    i = pl.program_id(0)
    j = lax.rem(i, _NBUF)
    uf = idx_ref[...].astype(jnp.float32)
    lane_iota = lax.broadcasted_iota(jnp.int32, (128, _NZ), 1).astype(jnp.float32)
    sub_iota = lax.broadcasted_iota(jnp.int32, (nsub, _NZ), 0)

    @pl.when(i >= _NBUF)
    def _():
        pltpu.make_async_copy(buf.at[j], out_hbm.at[pl.ds(0, _BLKR), :],
                              sems.at[j]).wait()

    for s in range(nsub):
        sel = (sub_iota == s).astype(jnp.float32)
        bc = lax.dot_general(uf, sel, (((0,), (0,)), ((), ())),
                             preferred_element_type=jnp.float32)
        oh = (bc == lane_iota).astype(jnp.float32)
        buf[j, pl.ds(s * 128, 128), :] = oh

    @pl.when(i < _TGRID - 1)
    def _():
        pltpu.make_async_copy(buf.at[j],
                              out_hbm.at[pl.ds(i * _BLKR, _BLKR), :],
                              sems.at[j]).start()

    @pl.when(i == _TGRID - 1)
    def _():
        jl = (_TGRID - 1) % _NBUF
        tail = pltpu.make_async_copy(
            buf.at[jl, pl.ds(0, _TAIL), :],
            out_hbm.at[pl.ds((_TGRID - 1) * _BLKR, _TAIL), :],
            sems.at[jl])
        tail.start()
        tail.wait()
        for k in range(_NBUF):
            if k != jl:
                pltpu.make_async_copy(buf.at[k],
                                      out_hbm.at[pl.ds(0, _BLKR), :],
                                      sems.at[k]).wait()


def kernel(zz, zmap):
    zz = zz.reshape(-1).astype(jnp.int32)
    zmap_p = jnp.zeros((128,), jnp.int32).at[:_NZ].set(zmap.astype(jnp.int32))
    idx = _sc_gather(zz, zmap_p)
    out = pl.pallas_call(
        _tc_body,
        grid=(_TGRID,),
        in_specs=[pl.BlockSpec((_BLKR // 128, 128), lambda i: (i, 0))],
        out_specs=pl.BlockSpec(memory_space=pltpu.HBM),
        out_shape=jax.ShapeDtypeStruct((_NROWS, _NZ), jnp.float32),
        scratch_shapes=[
            pltpu.VMEM((_NBUF, _BLKR, _NZ), jnp.float32),
            pltpu.SemaphoreType.DMA((_NBUF,)),
        ],
    )(idx.reshape(_NPAD // 128, 128))
    return out


# speedup vs baseline: 4.2756x; 1.0011x over previous
import functools

import jax
import jax.numpy as jnp
from jax import lax
from jax.experimental import pallas as pl
from jax.experimental.pallas import tpu as pltpu
from jax.experimental.pallas import tpu_sc as plsc

_NROWS = 1_000_000
_NZ = 100
_R = 400
_NCH = _NROWS // _R
_G = _R // 16
_NPAD = 1 << 20

_info = plsc.get_sparse_core_info()
_NC = _info.num_cores
_NW = _NC * _info.num_subcores
_NT_HI = -(-_NCH // _NW)
_NT_LO = _NCH // _NW
_N_HI = _NCH - _NT_LO * _NW
_ZZW = _NT_HI * _R


@functools.partial(
    pl.kernel,
    out_type=jax.ShapeDtypeStruct((_NPAD,), jnp.int32),
    mesh=plsc.VectorSubcoreMesh(core_axis_name="c", subcore_axis_name="s"),
    compiler_params=pltpu.CompilerParams(needs_layout_passes=False),
    scratch_types=[
        pltpu.VMEM((128,), jnp.int32),
        pltpu.VMEM((_ZZW,), jnp.int32),
        pltpu.VMEM((_ZZW,), jnp.int32),
    ],
)
def _sc_gather(zz_hbm, zmap_hbm, idx_hbm, zmap_v, zz_v, ibuf):
    w = lax.axis_index("s") * _NC + lax.axis_index("c")

    pltpu.sync_copy(zmap_hbm, zmap_v)

    is_hi = w < _N_HI
    start = jnp.where(is_hi, _NT_HI * w, _NT_LO * w + _N_HI)
    lo_words = _NT_LO * _R

    pltpu.sync_copy(zz_hbm.at[pl.ds(start * _R, lo_words)],
                    zz_v.at[pl.ds(0, lo_words)])

    @pl.when(is_hi)
    def _():
        pltpu.sync_copy(zz_hbm.at[pl.ds(start * _R + lo_words, _R)],
                        zz_v.at[pl.ds(lo_words, _R)])

    def _group(j, _):
        z = zz_v[pl.ds(j * 16, 16)]
        ibuf[pl.ds(j * 16, 16)] = plsc.load_gather(zmap_v, [z])
        return 0
    lax.fori_loop(0, lo_words // 16, _group, 0)

    @pl.when(is_hi)
    def _():
        for g in range(_G):
            z = zz_v[pl.ds(lo_words + g * 16, 16)]
            ibuf[pl.ds(lo_words + g * 16, 16)] = plsc.load_gather(zmap_v, [z])

    pltpu.sync_copy(ibuf.at[pl.ds(0, lo_words)],
                    idx_hbm.at[pl.ds(start * _R, lo_words)])

    @pl.when(is_hi)
    def _():
        pltpu.sync_copy(ibuf.at[pl.ds(lo_words, _R)],
                        idx_hbm.at[pl.ds(start * _R + lo_words, _R)])


_BLKR = 16384
_TGRID = -(-_NROWS // _BLKR)
_TAIL = _NROWS - (_TGRID - 1) * _BLKR
_NBUF = 4


def _tc_body(idx_ref, out_hbm, buf, sems):
    nsub = _BLKR // 128
    i = pl.program_id(0)
    j = lax.rem(i, _NBUF)
    uf = idx_ref[...].astype(jnp.float32)
    lane_iota = lax.broadcasted_iota(jnp.int32, (128, _NZ), 1).astype(jnp.float32)
    sub_iota = lax.broadcasted_iota(jnp.int32, (nsub, _NZ), 0)

    @pl.when(i >= _NBUF)
    def _():
        pltpu.make_async_copy(buf.at[j], out_hbm.at[pl.ds(0, _BLKR), :],
                              sems.at[j]).wait()

    for s in range(nsub):
        sel = (sub_iota == s).astype(jnp.float32)
        bc = lax.dot_general(uf, sel, (((0,), (0,)), ((), ())),
                             preferred_element_type=jnp.float32)
        oh = (bc == lane_iota).astype(jnp.float32)
        buf[j, pl.ds(s * 128, 128), :] = oh

    @pl.when(i < _TGRID - 1)
    def _():
        pltpu.make_async_copy(buf.at[j],
                              out_hbm.at[pl.ds(i * _BLKR, _BLKR), :],
                              sems.at[j]).start()

    @pl.when(i == _TGRID - 1)
    def _():
        jl = (_TGRID - 1) % _NBUF
        tail = pltpu.make_async_copy(
            buf.at[jl, pl.ds(0, _TAIL), :],
            out_hbm.at[pl.ds((_TGRID - 1) * _BLKR, _TAIL), :],
            sems.at[jl])
        tail.start()
        tail.wait()
        for k in range(_NBUF):
            if k != jl:
                pltpu.make_async_copy(buf.at[k],
                                      out_hbm.at[pl.ds(0, _BLKR), :],
                                      sems.at[k]).wait()


def kernel(zz, zmap):
    zz = zz.reshape(-1).astype(jnp.int32)
    zmap_p = jnp.zeros((128,), jnp.int32).at[:_NZ].set(zmap.astype(jnp.int32))
    idx = _sc_gather(zz, zmap_p)
    out = pl.pallas_call(
        _tc_body,
        grid=(_TGRID,),
        in_specs=[pl.BlockSpec((_BLKR // 128, 128), lambda i: (i, 0))],
        out_specs=pl.BlockSpec(memory_space=pltpu.HBM),
        out_shape=jax.ShapeDtypeStruct((_NROWS, _NZ), jnp.float32),
        scratch_shapes=[
            pltpu.VMEM((_NBUF, _BLKR, _NZ), jnp.float32),
            pltpu.SemaphoreType.DMA((_NBUF,)),
        ],
    )(idx.reshape(_NPAD // 128, 128))
    return out
